# Initial kernel scaffold; baseline (speedup 1.0000x reference)
import functools

import jax
import jax.numpy as jnp
from jax import lax
from jax.experimental import pallas as pl
from jax.experimental.pallas import tpu as pltpu
from jax.experimental.pallas import tpu_sc as plsc

N_WORKERS = 32
LANES = 16
T_CHUNK = 32
RSQRT_MAGIC = jnp.int32(0x5F3759DF)


def _rsqrt16(x):
    i = plsc.bitcast(x, jnp.int32)
    i = RSQRT_MAGIC - lax.shift_right_logical(i, 1)
    y = plsc.bitcast(i, jnp.float32)
    for _ in range(3):
        y = y * (1.5 - 0.5 * x * y * y)
    return y


def kernel(field_tables, position_table, ln_scale, ln_bias, input_ids):
    n_fields, vocab, hidden = field_tables.shape
    batch, seq, _ = input_ids.shape
    n_tok = batch * seq
    assert hidden == 64 and n_fields == 26
    assert n_tok % (N_WORKERS * T_CHUNK) == 0
    per_w = n_tok // N_WORKERS
    n_chunks = per_w // T_CHUNK
    n_idx = T_CHUNK * n_fields
    n_groups = n_idx // 64
    n_vregs = n_idx // LANES
    pat = 208

    flat_tables = field_tables.reshape(n_fields * vocab, hidden)
    ids_flat = input_ids.reshape(-1)
    offpat = (jnp.arange(pat, dtype=jnp.int32) % n_fields) * vocab

    hv = hidden // LANES

    @functools.partial(
        pl.kernel,
        out_type=jax.ShapeDtypeStruct((n_tok, hidden), jnp.float32),
        mesh=plsc.VectorSubcoreMesh(core_axis_name="c", subcore_axis_name="s"),
        scratch_types=[
            pltpu.VMEM((n_idx,), jnp.int32),
            pltpu.VMEM((n_groups, 64), jnp.int32),
            pltpu.VMEM((n_idx, hidden), jnp.float32),
            pltpu.VMEM((seq, hidden), jnp.float32),
            pltpu.VMEM((hidden,), jnp.float32),
            pltpu.VMEM((hidden,), jnp.float32),
            pltpu.VMEM((pat,), jnp.int32),
            pltpu.VMEM((T_CHUNK, hidden), jnp.float32),
            pltpu.SemaphoreType.DMA,
        ],
    )
    def emb_kernel(table_hbm, ids_hbm, pos_hbm, scale_hbm, bias_hbm,
                   off_hbm, out_hbm,
                   ids_v, idx_v, rows_v, pos_v, scale_v, bias_v, off_v,
                   out_v, sem):
        wid = lax.axis_index("c") * 16 + lax.axis_index("s")

        pltpu.sync_copy(pos_hbm.at[pl.ds(0, seq)], pos_v)
        pltpu.sync_copy(scale_hbm, scale_v)
        pltpu.sync_copy(bias_hbm, bias_v)
        pltpu.sync_copy(off_hbm, off_v)

        tok_w0 = wid * per_w

        def chunk_body(c, carry):
            tok0 = tok_w0 + c * T_CHUNK
            id0 = pl.multiple_of(tok0 * n_fields, 8)
            pltpu.sync_copy(ids_hbm.at[pl.ds(id0, n_idx)], ids_v)

            for v in range(n_vregs):
                g, col = v // 4, 16 * (v % 4)
                vec = ids_v[pl.ds(16 * v, 16)] + off_v[pl.ds(16 * (v % 13), 16)]
                idx_v[g, pl.ds(col, 16)] = vec

            copies = []
            for g in range(n_groups):
                cp = pltpu.async_copy(
                    table_hbm.at[idx_v.at[g]],
                    rows_v.at[pl.ds(64 * g, 64)],
                    sem,
                )
                copies.append(cp)
            for cp in copies:
                cp.wait()

            def tok_body(i, carry2):
                s = lax.rem(tok0 + i, seq)
                base = i * n_fields
                accs = [pos_v[s, pl.ds(16 * j, 16)] for j in range(hv)]
                for f in range(n_fields):
                    for j in range(hv):
                        accs[j] = accs[j] + rows_v[base + f, pl.ds(16 * j, 16)]
                tot = accs[0] + accs[1] + accs[2] + accs[3]
                mean = jnp.sum(tot) * (1.0 / hidden)
                d = [a - mean for a in accs]
                sq = d[0] * d[0] + d[1] * d[1] + d[2] * d[2] + d[3] * d[3]
                var = jnp.sum(sq) * (1.0 / hidden)
                rstd = _rsqrt16(jnp.zeros((16,), jnp.float32) + (var + 1e-12))
                for j in range(hv):
                    val = (d[j] * rstd * scale_v[pl.ds(16 * j, 16)]
                           + bias_v[pl.ds(16 * j, 16)])
                    out_v[i, pl.ds(16 * j, 16)] = val
                return carry2

            lax.fori_loop(0, T_CHUNK, tok_body, 0)
            pltpu.sync_copy(out_v, out_hbm.at[pl.ds(tok0, T_CHUNK)])
            return carry

        lax.fori_loop(0, n_chunks, chunk_body, 0)

    out = emb_kernel(flat_tables, ids_flat, position_table,
                     ln_scale, ln_bias, offpat)
    return out.reshape(batch, seq, hidden)


# speedup vs baseline: 2.1290x; 2.1290x over previous
import functools

import jax
import jax.numpy as jnp
from jax import lax
from jax.experimental import pallas as pl
from jax.experimental.pallas import tpu as pltpu
from jax.experimental.pallas import tpu_sc as plsc

N_WORKERS = 32
LANES = 16
T_CHUNK = 32
RSQRT_MAGIC = 0x5F3759DF


def _lane_sum16(x):
    lanes = lax.iota(jnp.int32, 16)
    for off in (1, 2, 4, 8):
        perm = lax.bitwise_xor(lanes, jnp.full((16,), off, jnp.int32))
        x = x + x.at[perm].get(mode="promise_in_bounds")
    return x


def _rsqrt16(x):
    i = plsc.bitcast(x, jnp.int32)
    i = jnp.full((16,), RSQRT_MAGIC, jnp.int32) - lax.shift_right_logical(i, 1)
    y = plsc.bitcast(i, jnp.float32)
    for _ in range(3):
        y = y * (1.5 - 0.5 * x * y * y)
    return y


def kernel(field_tables, position_table, ln_scale, ln_bias, input_ids):
    n_fields, vocab, hidden = field_tables.shape
    batch, seq, _ = input_ids.shape
    n_tok = batch * seq
    assert hidden == 64 and n_fields == 26
    assert n_tok % (N_WORKERS * T_CHUNK) == 0
    per_w = n_tok // N_WORKERS
    n_chunks = per_w // T_CHUNK
    n_idx = T_CHUNK * n_fields
    n_groups = n_idx // 64
    n_vregs = n_idx // LANES
    pat = 208

    flat_tables = field_tables.reshape(n_fields * vocab, hidden)
    ids_flat = input_ids.reshape(-1)
    offpat = (jnp.arange(pat, dtype=jnp.int32) % n_fields) * vocab

    hv = hidden // LANES

    @functools.partial(
        pl.kernel,
        out_type=jax.ShapeDtypeStruct((n_tok, hidden), jnp.float32),
        mesh=plsc.VectorSubcoreMesh(core_axis_name="c", subcore_axis_name="s"),
        compiler_params=pltpu.CompilerParams(
            needs_layout_passes=False, use_tc_tiling_on_sc=False),
        scratch_types=[
            pltpu.VMEM((n_idx,), jnp.int32),
            pltpu.VMEM((n_groups, 64), jnp.int32),
            pltpu.VMEM((n_idx, hidden), jnp.float32),
            pltpu.VMEM((seq, hidden), jnp.float32),
            pltpu.VMEM((hidden,), jnp.float32),
            pltpu.VMEM((hidden,), jnp.float32),
            pltpu.VMEM((pat,), jnp.int32),
            pltpu.VMEM((T_CHUNK, hidden), jnp.float32),
            pltpu.SemaphoreType.DMA,
        ],
    )
    def emb_kernel(table_hbm, ids_hbm, pos_hbm, scale_hbm, bias_hbm,
                   off_hbm, out_hbm,
                   ids_v, idx_v, rows_v, pos_v, scale_v, bias_v, off_v,
                   out_v, sem):
        wid = lax.axis_index("c") * 16 + lax.axis_index("s")

        pltpu.sync_copy(pos_hbm.at[pl.ds(0, seq)], pos_v)
        pltpu.sync_copy(scale_hbm, scale_v)
        pltpu.sync_copy(bias_hbm, bias_v)
        pltpu.sync_copy(off_hbm, off_v)

        tok_w0 = wid * per_w

        def chunk_body(c, carry):
            tok0 = tok_w0 + c * T_CHUNK
            id0 = pl.multiple_of(tok0 * n_fields, 8)
            pltpu.sync_copy(ids_hbm.at[pl.ds(id0, n_idx)], ids_v)

            for v in range(n_vregs):
                g, col = v // 4, 16 * (v % 4)
                vec = ids_v[pl.ds(16 * v, 16)] + off_v[pl.ds(16 * (v % 13), 16)]
                idx_v[g, pl.ds(col, 16)] = vec

            copies = []
            for g in range(n_groups):
                cp = pltpu.async_copy(
                    table_hbm.at[idx_v.at[g]],
                    rows_v.at[pl.ds(64 * g, 64)],
                    sem,
                )
                copies.append(cp)
            for cp in copies:
                cp.wait()

            def tok_body(i, carry2):
                s = lax.rem(tok0 + i, seq)
                base = i * n_fields
                accs = [pos_v[s, pl.ds(16 * j, 16)] for j in range(hv)]
                for f in range(n_fields):
                    for j in range(hv):
                        accs[j] = accs[j] + rows_v[base + f, pl.ds(16 * j, 16)]
                tot = accs[0] + accs[1] + accs[2] + accs[3]
                mean = _lane_sum16(tot) * (1.0 / hidden)
                d = [a - mean for a in accs]
                sq = d[0] * d[0] + d[1] * d[1] + d[2] * d[2] + d[3] * d[3]
                var = _lane_sum16(sq) * (1.0 / hidden)
                rstd = _rsqrt16(var + 1e-12)
                for j in range(hv):
                    val = (d[j] * rstd * scale_v[pl.ds(16 * j, 16)]
                           + bias_v[pl.ds(16 * j, 16)])
                    out_v[i, pl.ds(16 * j, 16)] = val
                return carry2

            lax.fori_loop(0, T_CHUNK, tok_body, 0)
            pltpu.sync_copy(out_v, out_hbm.at[pl.ds(tok0, T_CHUNK)])
            return carry

        lax.fori_loop(0, n_chunks, chunk_body, 0)

    out = emb_kernel(flat_tables, ids_flat, position_table,
                     ln_scale, ln_bias, offpat)
    return out.reshape(batch, seq, hidden)


# speedup vs baseline: 2.5258x; 1.1864x over previous
import functools

import jax
import jax.numpy as jnp
from jax import lax
from jax.experimental import pallas as pl
from jax.experimental.pallas import tpu as pltpu
from jax.experimental.pallas import tpu_sc as plsc

N_WORKERS = 32
LANES = 16
T_CHUNK = 32
RSQRT_MAGIC = 0x5F3759DF


def _lane_sum16(x):
    lanes = lax.iota(jnp.int32, 16)
    for off in (1, 2, 4, 8):
        perm = lax.bitwise_xor(lanes, jnp.full((16,), off, jnp.int32))
        x = x + x.at[perm].get(mode="promise_in_bounds")
    return x


def _rsqrt16(x):
    i = plsc.bitcast(x, jnp.int32)
    i = jnp.full((16,), RSQRT_MAGIC, jnp.int32) - lax.shift_right_logical(i, 1)
    y = plsc.bitcast(i, jnp.float32)
    for _ in range(3):
        y = y * (1.5 - 0.5 * x * y * y)
    return y


def kernel(field_tables, position_table, ln_scale, ln_bias, input_ids):
    n_fields, vocab, hidden = field_tables.shape
    batch, seq, _ = input_ids.shape
    n_tok = batch * seq
    assert hidden == 64 and n_fields == 26
    assert n_tok % (N_WORKERS * T_CHUNK) == 0
    per_w = n_tok // N_WORKERS
    n_chunks = per_w // T_CHUNK
    assert n_chunks % 2 == 0
    n_idx = T_CHUNK * n_fields
    n_groups = n_idx // 64
    n_vregs = n_idx // LANES
    pat = 208

    flat_tables = field_tables.reshape(n_fields * vocab, hidden)
    ids_flat = input_ids.reshape(-1)
    offpat = (jnp.arange(pat, dtype=jnp.int32) % n_fields) * vocab

    hv = hidden // LANES

    buf = lambda shp, dt: pltpu.VMEM(shp, dt)

    @functools.partial(
        pl.kernel,
        out_type=jax.ShapeDtypeStruct((n_tok, hidden), jnp.float32),
        mesh=plsc.VectorSubcoreMesh(core_axis_name="c", subcore_axis_name="s"),
        compiler_params=pltpu.CompilerParams(
            needs_layout_passes=False, use_tc_tiling_on_sc=False),
        scratch_types=[
            buf((n_idx,), jnp.int32), buf((n_idx,), jnp.int32),
            buf((n_groups, 64), jnp.int32), buf((n_groups, 64), jnp.int32),
            buf((n_idx, hidden), jnp.float32),
            buf((n_idx, hidden), jnp.float32),
            buf((T_CHUNK, hidden), jnp.float32),
            buf((T_CHUNK, hidden), jnp.float32),
            buf((seq, hidden), jnp.float32),
            buf((hidden,), jnp.float32),
            buf((hidden,), jnp.float32),
            buf((pat,), jnp.int32),
            pltpu.SemaphoreType.DMA,
            pltpu.SemaphoreType.DMA,
        ],
    )
    def emb_kernel(table_hbm, ids_hbm, pos_hbm, scale_hbm, bias_hbm,
                   off_hbm, out_hbm,
                   ids_v0, ids_v1, idx_v0, idx_v1, rows_v0, rows_v1,
                   out_v0, out_v1, pos_v, scale_v, bias_v, off_v,
                   sem0, sem1):
        wid = lax.axis_index("c") * 16 + lax.axis_index("s")

        pltpu.sync_copy(pos_hbm.at[pl.ds(0, seq)], pos_v)
        pltpu.sync_copy(scale_hbm, scale_v)
        pltpu.sync_copy(bias_hbm, bias_v)
        pltpu.sync_copy(off_hbm, off_v)

        tok_w0 = wid * per_w
        bufs = ((ids_v0, idx_v0, rows_v0, out_v0, sem0),
                (ids_v1, idx_v1, rows_v1, out_v1, sem1))

        def fire(c, k):
            ids_v, idx_v, rows_v, _, sem = bufs[k]
            tok0 = tok_w0 + c * T_CHUNK
            id0 = pl.multiple_of(tok0 * n_fields, 8)
            pltpu.sync_copy(ids_hbm.at[pl.ds(id0, n_idx)], ids_v)
            for v in range(n_vregs):
                g, col = v // 4, 16 * (v % 4)
                vec = ids_v[pl.ds(16 * v, 16)] + off_v[pl.ds(16 * (v % 13), 16)]
                idx_v[g, pl.ds(col, 16)] = vec
            for g in range(n_groups):
                pltpu.async_copy(
                    table_hbm.at[idx_v.at[g]],
                    rows_v.at[pl.ds(64 * g, 64)],
                    sem,
                )

        def compute(c, k):
            _, idx_v, rows_v, out_v, sem = bufs[k]
            tok0 = tok_w0 + c * T_CHUNK
            for g in range(n_groups):
                pltpu.make_async_copy(
                    table_hbm.at[idx_v.at[g]],
                    rows_v.at[pl.ds(64 * g, 64)],
                    sem,
                ).wait()

            def tok_body(i, carry2):
                s = lax.rem(tok0 + i, seq)
                base = i * n_fields
                accs = [pos_v[s, pl.ds(16 * j, 16)] for j in range(hv)]
                for f in range(n_fields):
                    for j in range(hv):
                        accs[j] = accs[j] + rows_v[base + f, pl.ds(16 * j, 16)]
                tot = accs[0] + accs[1] + accs[2] + accs[3]
                mean = _lane_sum16(tot) * (1.0 / hidden)
                d = [a - mean for a in accs]
                sq = d[0] * d[0] + d[1] * d[1] + d[2] * d[2] + d[3] * d[3]
                var = _lane_sum16(sq) * (1.0 / hidden)
                rstd = _rsqrt16(var + 1e-12)
                for j in range(hv):
                    val = (d[j] * rstd * scale_v[pl.ds(16 * j, 16)]
                           + bias_v[pl.ds(16 * j, 16)])
                    out_v[i, pl.ds(16 * j, 16)] = val
                return carry2

            lax.fori_loop(0, T_CHUNK, tok_body, 0)
            pltpu.sync_copy(out_v, out_hbm.at[pl.ds(tok0, T_CHUNK)])

        fire(0, 0)

        def pair_body(c2, carry):
            c = 2 * c2
            fire(c + 1, 1)
            compute(c, 0)

            @pl.when(c + 2 < n_chunks)
            def _():
                fire(c + 2, 0)

            compute(c + 1, 1)
            return carry

        lax.fori_loop(0, n_chunks // 2, pair_body, 0)

    out = emb_kernel(flat_tables, ids_flat, position_table,
                     ln_scale, ln_bias, offpat)
    return out.reshape(batch, seq, hidden)


# speedup vs baseline: 2.5266x; 1.0003x over previous
import functools

import jax
import jax.numpy as jnp
from jax import lax
from jax.experimental import pallas as pl
from jax.experimental.pallas import tpu as pltpu
from jax.experimental.pallas import tpu_sc as plsc

N_WORKERS = 32
LANES = 16
T_CHUNK = 32
RSQRT_MAGIC = 0x5F3759DF


def _lane_sum16(x):
    lanes = lax.iota(jnp.int32, 16)
    for off in (1, 2, 4, 8):
        perm = lax.bitwise_xor(lanes, jnp.full((16,), off, jnp.int32))
        x = x + x.at[perm].get(mode="promise_in_bounds")
    return x


def _rsqrt16(x):
    i = plsc.bitcast(x, jnp.int32)
    i = jnp.full((16,), RSQRT_MAGIC, jnp.int32) - lax.shift_right_logical(i, 1)
    y = plsc.bitcast(i, jnp.float32)
    for _ in range(3):
        y = y * (1.5 - 0.5 * x * y * y)
    return y


def kernel(field_tables, position_table, ln_scale, ln_bias, input_ids):
    n_fields, vocab, hidden = field_tables.shape
    batch, seq, _ = input_ids.shape
    n_tok = batch * seq
    assert hidden == 64 and n_fields == 26
    assert n_tok % (N_WORKERS * T_CHUNK) == 0
    per_w = n_tok // N_WORKERS
    n_chunks = per_w // T_CHUNK
    assert n_chunks % 2 == 0
    n_idx = T_CHUNK * n_fields
    n_groups = n_idx // 64

    flat_tables = field_tables.reshape(n_fields * vocab, hidden)
    offsets = jnp.arange(n_fields, dtype=jnp.int32) * vocab
    ids2d = (input_ids + offsets[None, None, :]).reshape(-1, 64)

    hv = hidden // LANES

    buf = lambda shp, dt: pltpu.VMEM(shp, dt)

    @functools.partial(
        pl.kernel,
        out_type=jax.ShapeDtypeStruct((n_tok, hidden), jnp.float32),
        mesh=plsc.VectorSubcoreMesh(core_axis_name="c", subcore_axis_name="s"),
        compiler_params=pltpu.CompilerParams(
            needs_layout_passes=False, use_tc_tiling_on_sc=False),
        scratch_types=[
            buf((n_groups, 64), jnp.int32), buf((n_groups, 64), jnp.int32),
            buf((n_idx, hidden), jnp.float32),
            buf((n_idx, hidden), jnp.float32),
            buf((T_CHUNK, hidden), jnp.float32),
            buf((T_CHUNK, hidden), jnp.float32),
            buf((seq * hidden,), jnp.float32),
            buf((hidden,), jnp.float32),
            buf((hidden,), jnp.float32),
            pltpu.SemaphoreType.DMA,
            pltpu.SemaphoreType.DMA,
        ],
    )
    def emb_kernel(table_hbm, ids_hbm, pos_hbm, scale_hbm, bias_hbm,
                   out_hbm,
                   idx_v0, idx_v1, rows_v0, rows_v1,
                   out_v0, out_v1, pos_v, scale_v, bias_v,
                   sem0, sem1):
        wid = lax.axis_index("c") * 16 + lax.axis_index("s")

        pltpu.sync_copy(pos_hbm.at[pl.ds(0, seq * hidden)], pos_v)
        pltpu.sync_copy(scale_hbm, scale_v)
        pltpu.sync_copy(bias_hbm, bias_v)

        tok_w0 = wid * per_w
        bufs = ((idx_v0, rows_v0, out_v0, sem0),
                (idx_v1, rows_v1, out_v1, sem1))

        def fire(c, k):
            idx_v, rows_v, _, sem = bufs[k]
            tok0 = tok_w0 + c * T_CHUNK
            row0 = tok0 * n_fields // 64
            pltpu.sync_copy(ids_hbm.at[pl.ds(row0, n_groups)], idx_v)
            for g in range(n_groups):
                pltpu.async_copy(
                    table_hbm.at[idx_v.at[g]],
                    rows_v.at[pl.ds(64 * g, 64)],
                    sem,
                )

        def compute(c, k):
            idx_v, rows_v, out_v, sem = bufs[k]
            tok0 = tok_w0 + c * T_CHUNK
            for g in range(n_groups):
                pltpu.make_async_copy(
                    table_hbm.at[idx_v.at[g]],
                    rows_v.at[pl.ds(64 * g, 64)],
                    sem,
                ).wait()

            def tok_body(i, carry2):
                s = lax.rem(tok0 + i, seq)
                base = i * n_fields
                accs = [pos_v[pl.ds(s * hidden + 16 * j, 16)]
                        for j in range(hv)]
                for f in range(n_fields):
                    for j in range(hv):
                        accs[j] = accs[j] + rows_v[base + f, pl.ds(16 * j, 16)]
                tot = accs[0] + accs[1] + accs[2] + accs[3]
                mean = _lane_sum16(tot) * (1.0 / hidden)
                d = [a - mean for a in accs]
                sq = d[0] * d[0] + d[1] * d[1] + d[2] * d[2] + d[3] * d[3]
                var = _lane_sum16(sq) * (1.0 / hidden)
                rstd = _rsqrt16(var + 1e-12)
                for j in range(hv):
                    val = (d[j] * rstd * scale_v[pl.ds(16 * j, 16)]
                           + bias_v[pl.ds(16 * j, 16)])
                    out_v[i, pl.ds(16 * j, 16)] = val
                return carry2

            lax.fori_loop(0, T_CHUNK, tok_body, 0)
            pltpu.sync_copy(out_v, out_hbm.at[pl.ds(tok0, T_CHUNK)])

        fire(0, 0)

        def pair_body(c2, carry):
            c = 2 * c2
            fire(c + 1, 1)
            compute(c, 0)

            @pl.when(c + 2 < n_chunks)
            def _():
                fire(c + 2, 0)

            compute(c + 1, 1)
            return carry

        lax.fori_loop(0, n_chunks // 2, pair_body, 0)

    pos_flat = (position_table * jnp.float32(1.0)).reshape(-1)
    out = emb_kernel(flat_tables, ids2d, pos_flat, ln_scale, ln_bias)
    return out.reshape(batch, seq, hidden)
